# baseline (speedup 1.0000x reference)
import jax
import jax.numpy as jnp
from jax.experimental import pallas as pl
from jax.experimental.pallas import tpu as pltpu

N, D, E, H_R, H_E = 4096, 1024, 16, 64, 128
TBLK = 2048
NPAIR = E // 2


def _moe_kernel(x_ref, rw1_ref, rb1_ref, rw2_ref, rb2_ref,
                ew1_ref, eb1_ref, ew2_ref, eb2f_ref,
                y_ref, w_ref, wtop_ref):
    j = pl.program_id(1)

    @pl.when(j == 0)
    def _router():
        xb = x_ref[...]
        hr = jnp.maximum(
            jnp.dot(xb, rw1_ref[...], preferred_element_type=jnp.float32)
            + rb1_ref[...][None, :], 0.0)
        logits = (jnp.dot(hr, rw2_ref[...], preferred_element_type=jnp.float32)
                  + rb2_ref[...][None, :])
        logits = logits - jnp.max(logits, axis=-1, keepdims=True)
        ew = jnp.exp(logits)
        w = ew / jnp.sum(ew, axis=-1, keepdims=True)
        w_ref[...] = w
        cols = jax.lax.broadcasted_iota(jnp.int32, w.shape, 1)
        i1 = jnp.argmax(w, axis=-1)[:, None]
        w2 = jnp.where(cols == i1, -jnp.inf, w)
        i2 = jnp.argmax(w2, axis=-1)[:, None]
        mask = (cols == i1) | (cols == i2)
        wt = jnp.where(mask, w, 0.0)
        wtop = wt / (jnp.sum(wt, axis=-1, keepdims=True) + 1e-8)
        wtop_ref[...] = wtop

    xb = x_ref[...].astype(jnp.bfloat16)
    h = jnp.tanh(
        jnp.dot(xb, ew1_ref[0], preferred_element_type=jnp.float32)
        + eb1_ref[0])
    wt = wtop_ref[...]
    ecols = jax.lax.broadcasted_iota(jnp.int32, wt.shape, 1)
    wa = jnp.sum(jnp.where(ecols == 2 * j, wt, 0.0), axis=-1, keepdims=True)
    wb = jnp.sum(jnp.where(ecols == 2 * j + 1, wt, 0.0), axis=-1,
                 keepdims=True)
    hcols = jax.lax.broadcasted_iota(jnp.int32, h.shape, 1)
    hs = (h * jnp.where(hcols < H_E, wa, wb)).astype(jnp.bfloat16)
    contrib = (jnp.dot(hs, ew2_ref[0], preferred_element_type=jnp.float32)
               + wa * eb2f_ref[0, 0] + wb * eb2f_ref[0, 1])

    @pl.when(j == 0)
    def _init():
        y_ref[...] = contrib

    @pl.when(j != 0)
    def _acc():
        y_ref[...] += contrib


@jax.jit
def kernel(x, rw1, rb1, rw2, rb2, ew1, eb1, ew2, eb2):
    n_tblk = N // TBLK
    grid = (n_tblk, NPAIR)
    ew1p = ew1.reshape(NPAIR, 2, D, H_E).transpose(0, 2, 1, 3).reshape(
        NPAIR, D, 2 * H_E).astype(jnp.bfloat16)
    eb1p = eb1.reshape(NPAIR, 1, 2 * H_E)
    ew2p = ew2.reshape(NPAIR, 2 * H_E, D).astype(jnp.bfloat16)
    y, w = pl.pallas_call(
        _moe_kernel,
        grid=grid,
        in_specs=[
            pl.BlockSpec((TBLK, D), lambda i, j: (i, 0)),
            pl.BlockSpec((D, H_R), lambda i, j: (0, 0)),
            pl.BlockSpec((H_R,), lambda i, j: (0,)),
            pl.BlockSpec((H_R, E), lambda i, j: (0, 0)),
            pl.BlockSpec((E,), lambda i, j: (0,)),
            pl.BlockSpec((1, D, 2 * H_E), lambda i, j: (j, 0, 0)),
            pl.BlockSpec((1, 1, 2 * H_E), lambda i, j: (j, 0, 0)),
            pl.BlockSpec((1, 2 * H_E, D), lambda i, j: (j, 0, 0)),
            pl.BlockSpec((1, 2, D), lambda i, j: (j, 0, 0)),
        ],
        out_specs=[
            pl.BlockSpec((TBLK, D), lambda i, j: (i, 0)),
            pl.BlockSpec((TBLK, E), lambda i, j: (i, 0)),
        ],
        out_shape=[
            jax.ShapeDtypeStruct((N, D), jnp.float32),
            jax.ShapeDtypeStruct((N, E), jnp.float32),
        ],
        scratch_shapes=[pltpu.VMEM((TBLK, E), jnp.float32)],
        compiler_params=pltpu.CompilerParams(
            dimension_semantics=("parallel", "arbitrary")),
    )(x, rw1, rb1, rw2, rb2, ew1p, eb1p, ew2p,
      eb2.reshape(NPAIR, 2, D))
    return (y, w)


# speedup vs baseline: 1.0886x; 1.0886x over previous
import jax
import jax.numpy as jnp
from jax.experimental import pallas as pl
from jax.experimental.pallas import tpu as pltpu

N, D, E, H_R, H_E = 4096, 1024, 16, 64, 128
TBLK = 4096
NPAIR = E // 2


def _moe_kernel(x_ref, rw1_ref, rb1_ref, rw2_ref, rb2_ref,
                ew1_ref, eb1_ref, ew2_ref, eb2f_ref,
                y_ref, w_ref, wtop_ref):
    j = pl.program_id(1)

    @pl.when(j == 0)
    def _router():
        xb = x_ref[...]
        hr = jnp.maximum(
            jnp.dot(xb, rw1_ref[...], preferred_element_type=jnp.float32)
            + rb1_ref[...][None, :], 0.0)
        logits = (jnp.dot(hr, rw2_ref[...], preferred_element_type=jnp.float32)
                  + rb2_ref[...][None, :])
        logits = logits - jnp.max(logits, axis=-1, keepdims=True)
        ew = jnp.exp(logits)
        w = ew / jnp.sum(ew, axis=-1, keepdims=True)
        w_ref[...] = w
        cols = jax.lax.broadcasted_iota(jnp.int32, w.shape, 1)
        i1 = jnp.argmax(w, axis=-1)[:, None]
        w2 = jnp.where(cols == i1, -jnp.inf, w)
        i2 = jnp.argmax(w2, axis=-1)[:, None]
        mask = (cols == i1) | (cols == i2)
        wt = jnp.where(mask, w, 0.0)
        wtop = wt / (jnp.sum(wt, axis=-1, keepdims=True) + 1e-8)
        wtop_ref[...] = wtop
        y_ref[...] = jnp.dot(wtop, eb2f_ref[...],
                             preferred_element_type=jnp.float32)

    xb = x_ref[...].astype(jnp.bfloat16)
    h = jnp.tanh(
        jnp.dot(xb, ew1_ref[0], preferred_element_type=jnp.float32)
        + eb1_ref[0])
    wt = wtop_ref[...]
    ecols = jax.lax.broadcasted_iota(jnp.int32, wt.shape, 1)
    wa = jnp.sum(jnp.where(ecols == 2 * j, wt, 0.0), axis=-1, keepdims=True)
    wb = jnp.sum(jnp.where(ecols == 2 * j + 1, wt, 0.0), axis=-1,
                 keepdims=True)
    hcols = jax.lax.broadcasted_iota(jnp.int32, h.shape, 1)
    hs = (h * jnp.where(hcols < H_E, wa, wb)).astype(jnp.bfloat16)
    y_ref[...] += jnp.dot(hs, ew2_ref[0], preferred_element_type=jnp.float32)


@jax.jit
def kernel(x, rw1, rb1, rw2, rb2, ew1, eb1, ew2, eb2):
    n_tblk = N // TBLK
    grid = (n_tblk, NPAIR)
    ew1p = ew1.reshape(NPAIR, 2, D, H_E).transpose(0, 2, 1, 3).reshape(
        NPAIR, D, 2 * H_E).astype(jnp.bfloat16)
    eb1p = eb1.reshape(NPAIR, 1, 2 * H_E)
    ew2p = ew2.reshape(NPAIR, 2 * H_E, D).astype(jnp.bfloat16)
    y, w = pl.pallas_call(
        _moe_kernel,
        grid=grid,
        in_specs=[
            pl.BlockSpec((TBLK, D), lambda i, j: (i, 0)),
            pl.BlockSpec((D, H_R), lambda i, j: (0, 0)),
            pl.BlockSpec((H_R,), lambda i, j: (0,)),
            pl.BlockSpec((H_R, E), lambda i, j: (0, 0)),
            pl.BlockSpec((E,), lambda i, j: (0,)),
            pl.BlockSpec((1, D, 2 * H_E), lambda i, j: (j, 0, 0)),
            pl.BlockSpec((1, 1, 2 * H_E), lambda i, j: (j, 0, 0)),
            pl.BlockSpec((1, 2 * H_E, D), lambda i, j: (j, 0, 0)),
            pl.BlockSpec((E, D), lambda i, j: (0, 0)),
        ],
        out_specs=[
            pl.BlockSpec((TBLK, D), lambda i, j: (i, 0)),
            pl.BlockSpec((TBLK, E), lambda i, j: (i, 0)),
        ],
        out_shape=[
            jax.ShapeDtypeStruct((N, D), jnp.float32),
            jax.ShapeDtypeStruct((N, E), jnp.float32),
        ],
        scratch_shapes=[pltpu.VMEM((TBLK, E), jnp.float32)],
        compiler_params=pltpu.CompilerParams(
            dimension_semantics=("parallel", "arbitrary")),
    )(x, rw1, rb1, rw2, rb2, ew1p, eb1p, ew2p, eb2)
    return (y, w)


# speedup vs baseline: 1.6772x; 1.5407x over previous
import jax
import jax.numpy as jnp
from jax.experimental import pallas as pl
from jax.experimental.pallas import tpu as pltpu

N, D, E, H_R, H_E = 4096, 1024, 16, 64, 128
TBLK = 1024
HF = E * H_E
KX = HF + E


def _router_kernel(x_ref, rw1_ref, rb1_ref, rw2_ref, rb2_ref,
                   w_ref, wtop_ref):
    xb = x_ref[...]
    hr = jnp.maximum(
        jnp.dot(xb, rw1_ref[...], preferred_element_type=jnp.float32)
        + rb1_ref[...][None, :], 0.0)
    logits = (jnp.dot(hr, rw2_ref[...], preferred_element_type=jnp.float32)
              + rb2_ref[...][None, :])
    logits = logits - jnp.max(logits, axis=-1, keepdims=True)
    ew = jnp.exp(logits)
    w = ew / jnp.sum(ew, axis=-1, keepdims=True)
    w_ref[...] = w
    cols = jax.lax.broadcasted_iota(jnp.int32, w.shape, 1)
    i1 = jnp.argmax(w, axis=-1)[:, None]
    w2 = jnp.where(cols == i1, -jnp.inf, w)
    i2 = jnp.argmax(w2, axis=-1)[:, None]
    mask = (cols == i1) | (cols == i2)
    wt = jnp.where(mask, w, 0.0)
    wtop_ref[...] = wt / (jnp.sum(wt, axis=-1, keepdims=True) + 1e-8)


def _expert_kernel(x_ref, wtop_ref, ew1_ref, eb1_ref, ew2_ref,
                   y_ref, hs_ref):
    xb = x_ref[...].astype(jnp.bfloat16)
    pre = (jnp.dot(xb, ew1_ref[...], preferred_element_type=jnp.float32)
           + eb1_ref[...])
    h = jnp.tanh(pre)
    wt = wtop_ref[...]
    gates = jnp.broadcast_to(wt[:, :, None], (TBLK, E, H_E)).reshape(TBLK, HF)
    hs_ref[:, :HF] = (h * gates).astype(jnp.bfloat16)
    hs_ref[:, HF:] = wt.astype(jnp.bfloat16)
    y_ref[...] = jnp.dot(hs_ref[...], ew2_ref[...],
                         preferred_element_type=jnp.float32)


@jax.jit
def kernel(x, rw1, rb1, rw2, rb2, ew1, eb1, ew2, eb2):
    w, wtop = pl.pallas_call(
        _router_kernel,
        grid=(1,),
        in_specs=[
            pl.BlockSpec((N, D), lambda i: (0, 0)),
            pl.BlockSpec((D, H_R), lambda i: (0, 0)),
            pl.BlockSpec((H_R,), lambda i: (0,)),
            pl.BlockSpec((H_R, E), lambda i: (0, 0)),
            pl.BlockSpec((E,), lambda i: (0,)),
        ],
        out_specs=[
            pl.BlockSpec((N, E), lambda i: (0, 0)),
            pl.BlockSpec((N, E), lambda i: (0, 0)),
        ],
        out_shape=[
            jax.ShapeDtypeStruct((N, E), jnp.float32),
            jax.ShapeDtypeStruct((N, E), jnp.float32),
        ],
    )(x, rw1, rb1, rw2, rb2)

    ew1f = ew1.transpose(1, 0, 2).reshape(D, HF).astype(jnp.bfloat16)
    eb1f = eb1.reshape(1, HF)
    ew2f = jnp.concatenate(
        [ew2.reshape(HF, D), eb2], axis=0).astype(jnp.bfloat16)

    y = pl.pallas_call(
        _expert_kernel,
        grid=(N // TBLK,),
        in_specs=[
            pl.BlockSpec((TBLK, D), lambda i: (i, 0)),
            pl.BlockSpec((TBLK, E), lambda i: (i, 0)),
            pl.BlockSpec((D, HF), lambda i: (0, 0)),
            pl.BlockSpec((1, HF), lambda i: (0, 0)),
            pl.BlockSpec((KX, D), lambda i: (0, 0)),
        ],
        out_specs=pl.BlockSpec((TBLK, D), lambda i: (i, 0)),
        out_shape=jax.ShapeDtypeStruct((N, D), jnp.float32),
        scratch_shapes=[pltpu.VMEM((TBLK, KX), jnp.bfloat16)],
        compiler_params=pltpu.CompilerParams(
            dimension_semantics=("parallel",)),
    )(x, wtop, ew1f, eb1f, ew2f)
    return (y, w)
